# baseline (speedup 1.0000x reference)
import functools

import jax
import jax.numpy as jnp
from jax import lax
from jax.experimental import pallas as pl
from jax.experimental.pallas import tpu as pltpu
from jax.experimental.pallas import tpu_sc as plsc

PAD_ID = 0
DIM = 64
LANES = 16
CHUNK = 128
NUM_CORES = 2
NUM_SUBCORES = 16
NUM_WORKERS = NUM_CORES * NUM_SUBCORES


@functools.partial(jax.jit, static_argnames=("n_chunks",))
def _sc_embedding_lookup(idx3d, table, *, n_chunks):
    n_total = NUM_WORKERS * n_chunks * CHUNK
    mesh = plsc.VectorSubcoreMesh(
        core_axis_name="c", subcore_axis_name="s",
        num_cores=NUM_CORES, num_subcores=NUM_SUBCORES,
    )

    @functools.partial(
        pl.kernel,
        out_type=jax.ShapeDtypeStruct((n_total, DIM), jnp.float32),
        mesh=mesh,
        scratch_types=[
            pltpu.VMEM((n_chunks, CHUNK), jnp.int32),
            pltpu.VMEM((CHUNK, DIM), jnp.float32),
            pltpu.SemaphoreType.DMA,
        ],
        compiler_params=pltpu.CompilerParams(
            needs_layout_passes=False, use_tc_tiling_on_sc=False
        ),
    )
    def body(idx_hbm, table_hbm, out_hbm, idx_v, rows_v, gsem):
        wid = lax.axis_index("s") * NUM_CORES + lax.axis_index("c")
        base = wid * (n_chunks * CHUNK)
        pltpu.sync_copy(idx_hbm.at[wid], idx_v)

        def chunk_body(g, carry):
            pltpu.async_copy(table_hbm.at[idx_v.at[g]], rows_v, gsem).wait()

            acc = idx_v[g, pl.ds(0, LANES)]
            for i in range(1, CHUNK // LANES):
                acc = jnp.minimum(acc, idx_v[g, pl.ds(i * LANES, LANES)])
            n_pad = plsc.all_reduce_population_count(acc == PAD_ID)
            has_pad = n_pad[0] > 0

            @pl.when(has_pad)
            def _fixup():
                def row_body(r, c):
                    ival = plsc.load_gather(
                        idx_v,
                        [jnp.full((LANES,), g, jnp.int32),
                         jnp.full((LANES,), r, jnp.int32)],
                    )
                    m = jnp.where(ival == PAD_ID, 0.0, 1.0).astype(jnp.float32)
                    for k in range(DIM // LANES):
                        sl = pl.ds(k * LANES, LANES)
                        rows_v[r, sl] = rows_v[r, sl] * m
                    return c

                lax.fori_loop(0, CHUNK, row_body, 0)

            pltpu.sync_copy(rows_v, out_hbm.at[pl.ds(base + g * CHUNK, CHUNK)])
            return carry

        lax.fori_loop(0, n_chunks, chunk_body, 0)

    return body(idx3d, table)


def kernel(token_indices, table):
    b, t = token_indices.shape
    n = b * t
    n_chunks = n // (NUM_WORKERS * CHUNK)
    idx3d = token_indices.astype(jnp.int32).reshape(NUM_WORKERS, n_chunks, CHUNK)
    out = _sc_embedding_lookup(idx3d, table, n_chunks=n_chunks)
    return out.reshape(b, t, table.shape[1])


# speedup vs baseline: 1.1123x; 1.1123x over previous
import functools

import jax
import jax.numpy as jnp
from jax import lax
from jax.experimental import pallas as pl
from jax.experimental.pallas import tpu as pltpu
from jax.experimental.pallas import tpu_sc as plsc

PAD_ID = 0
DIM = 64
LANES = 16
CHUNK = 128
NBUF = 4
DIST = 2
NUM_CORES = 2
NUM_SUBCORES = 16
NUM_WORKERS = NUM_CORES * NUM_SUBCORES


@functools.partial(jax.jit, static_argnames=("n_chunks",))
def _sc_embedding_lookup(idx3d, table, *, n_chunks):
    n_total = NUM_WORKERS * n_chunks * CHUNK
    assert n_chunks % NBUF == 0
    mesh = plsc.VectorSubcoreMesh(
        core_axis_name="c", subcore_axis_name="s",
        num_cores=NUM_CORES, num_subcores=NUM_SUBCORES,
    )

    @functools.partial(
        pl.kernel,
        out_type=jax.ShapeDtypeStruct((n_total, DIM), jnp.float32),
        mesh=mesh,
        scratch_types=[
            pltpu.VMEM((n_chunks, CHUNK), jnp.int32),
            pltpu.VMEM((NBUF, CHUNK, DIM), jnp.float32),
            [pltpu.SemaphoreType.DMA] * NBUF,
            [pltpu.SemaphoreType.DMA] * NBUF,
        ],
        compiler_params=pltpu.CompilerParams(
            needs_layout_passes=False, use_tc_tiling_on_sc=False
        ),
    )
    def body(idx_hbm, table_hbm, out_hbm, idx_v, rows_v, gsems, ssems):
        wid = lax.axis_index("s") * NUM_CORES + lax.axis_index("c")
        base = wid * (n_chunks * CHUNK)
        pltpu.sync_copy(idx_hbm.at[wid], idx_v)

        def gather(g, b):
            return pltpu.make_async_copy(
                table_hbm.at[idx_v.at[g]], rows_v.at[b], gsems[b]
            )

        def store(g, b):
            return pltpu.make_async_copy(
                rows_v.at[b],
                out_hbm.at[pl.ds(base + g * CHUNK, CHUNK)],
                ssems[b],
            )

        for b in range(DIST):
            gather(b, b).start()

        def chunk_step(g, b):
            gather(g, b).wait()

            acc = idx_v[g, pl.ds(0, LANES)]
            for i in range(1, CHUNK // LANES):
                acc = jnp.minimum(acc, idx_v[g, pl.ds(i * LANES, LANES)])
            n_pad = plsc.all_reduce_population_count(acc == PAD_ID)

            @pl.when(n_pad[0] > 0)
            def _fixup():
                def row_body(r, c):
                    ival = plsc.load_gather(
                        idx_v,
                        [jnp.full((LANES,), g, jnp.int32),
                         jnp.full((LANES,), r, jnp.int32)],
                    )
                    m = jnp.where(ival == PAD_ID, 0.0, 1.0).astype(jnp.float32)
                    for k in range(DIM // LANES):
                        sl = pl.ds(k * LANES, LANES)
                        rows_v[b, r, sl] = rows_v[b, r, sl] * m
                    return c

                lax.fori_loop(0, CHUNK, row_body, 0)

            store(g, b).start()

            bn = (b + DIST) % NBUF
            gn = g + DIST

            @pl.when(gn - NBUF >= 0)
            def _drain():
                store(gn - NBUF, bn).wait()

            @pl.when(gn < n_chunks)
            def _prefetch():
                gather(gn, bn).start()

        def outer(j, carry):
            for b in range(NBUF):
                chunk_step(j * NBUF + b, b)
            return carry

        lax.fori_loop(0, n_chunks // NBUF, outer, 0)

        for g in range(n_chunks - DIST, n_chunks):
            store(g, g % NBUF).wait()

    return body(idx3d, table)


def kernel(token_indices, table):
    b, t = token_indices.shape
    n = b * t
    n_chunks = n // (NUM_WORKERS * CHUNK)
    idx3d = token_indices.astype(jnp.int32).reshape(NUM_WORKERS, n_chunks, CHUNK)
    out = _sc_embedding_lookup(idx3d, table, n_chunks=n_chunks)
    return out.reshape(b, t, table.shape[1])
